# baseline (speedup 1.0000x reference)
import functools

import jax
import jax.numpy as jnp
from jax import lax
from jax.experimental import pallas as pl
from jax.experimental.pallas import tpu as pltpu
from jax.experimental.pallas import tpu_sc as plsc

NC = 2
NS = 16
NW = NC * NS

CH = 1024
NBUF = 2


@functools.partial(jax.jit, static_argnums=(2, 3))
def _embedding_gather(tokens_flat, table, b_per_w, n_chunks):
    B = tokens_flat.shape[0]
    Vp, D = table.shape
    v_per_s = Vp // NS

    mesh = plsc.VectorSubcoreMesh(core_axis_name="c", subcore_axis_name="s")

    @functools.partial(
        pl.kernel,
        out_type=jax.ShapeDtypeStruct((B, D), jnp.float32),
        mesh=mesh,
        scratch_types=[
            pltpu.VMEM((NBUF, CH), jnp.int32),
            pltpu.VMEM((NBUF, CH, D), jnp.float32),
            pltpu.VMEM_SHARED((Vp, D), jnp.float32),
            pltpu.SemaphoreType.DMA((NBUF,)),
            pltpu.SemaphoreType.DMA((NBUF,)),
            pltpu.SemaphoreType.DMA((NBUF,)),
        ],
        compiler_params=pltpu.CompilerParams(use_tc_tiling_on_sc=False),
    )
    def k(tok_hbm, table_hbm, out_hbm, idx_v, rows_v, table_sh,
          sem_i, sem_g, sem_o):
        sid = lax.axis_index("s")
        wid = sid * NC + lax.axis_index("c")
        base = wid * b_per_w

        pltpu.sync_copy(
            table_hbm.at[pl.ds(sid * v_per_s, v_per_s)],
            table_sh.at[pl.ds(sid * v_per_s, v_per_s)],
        )
        plsc.subcore_barrier()

        def idx_copy(c, b):
            return pltpu.make_async_copy(
                tok_hbm.at[pl.ds(base + c * CH, CH)], idx_v.at[b], sem_i.at[b]
            )

        def out_copy(c, b):
            return pltpu.make_async_copy(
                rows_v.at[b], out_hbm.at[pl.ds(base + c * CH, CH)], sem_o.at[b]
            )

        for b in range(NBUF):
            idx_copy(b, b).start()

        def body(it, carry):
            for b in range(NBUF):
                c = it * NBUF + b
                idx_copy(c, b).wait()

                @pl.when(it > 0)
                def _():
                    out_copy(c - NBUF, b).wait()

                pltpu.async_copy(
                    table_sh.at[idx_v.at[b]],
                    rows_v.at[b],
                    sem_g.at[b],
                ).wait()

                out_copy(c, b).start()

                @pl.when(c + NBUF < n_chunks)
                def _():
                    idx_copy(c + NBUF, b).start()

            return carry

        lax.fori_loop(0, n_chunks // NBUF, body, 0)

        for b in range(NBUF):
            out_copy(n_chunks - NBUF + b, b).wait()

    return k(tokens_flat, table)


def kernel(tokens, table):
    B0, S = tokens.shape
    V, D = table.shape
    B = B0 * S
    b_per_w = B // NW
    n_chunks = b_per_w // CH
    assert B % NW == 0 and b_per_w % (CH * NBUF) == 0

    Vp = ((V + 8 * NS - 1) // (8 * NS)) * (8 * NS)
    table_p = jnp.pad(table, ((0, Vp - V), (0, 0)))

    out = _embedding_gather(tokens.reshape(B), table_p, b_per_w, n_chunks)
    return out.reshape(B0, S, D)
